# Initial kernel scaffold; baseline (speedup 1.0000x reference)
import functools

import jax
import jax.numpy as jnp
from jax import lax
from jax.experimental import pallas as pl
from jax.experimental.pallas import tpu as pltpu
from jax.experimental.pallas import tpu_sc as plsc

_N = 10000
_E = 320000
_R = 8
_B = 4

_NC = 2
_NS = 16
_NW = _NC * _NS
_EPW = _E // _NW
_K = 80
_NCHUNK = _EPW // _K
_RPT = _N // _NS

_BM = 1000


def _emit_transform(h, w_ref, c_ref, lw_ref, t_ref, loop_ref):
    bases = [
        jnp.dot(h, w_ref[b], preferred_element_type=jnp.float32)
        for b in range(_B)
    ]
    for r in range(_R):
        acc = c_ref[r, 0] * bases[0]
        for b in range(1, _B):
            acc = acc + c_ref[r, b] * bases[b]
        t_ref[r] = acc
    loop_ref[...] = jnp.dot(h, lw_ref[...], preferred_element_type=jnp.float32)


def _xform_first_body(x_ref, w_ref, c_ref, lw_ref, t_ref, loop_ref):
    _emit_transform(x_ref[...], w_ref, c_ref, lw_ref, t_ref, loop_ref)


def _xform_mid_body(acc_ref, lp_ref, b_ref, w_ref, c_ref, lw_ref, t_ref, loop_ref):
    h = jnp.maximum(acc_ref[0] + acc_ref[1] + lp_ref[...] + b_ref[...], 0.0)
    _emit_transform(h, w_ref, c_ref, lw_ref, t_ref, loop_ref)


def _final_body(acc_ref, lp_ref, b_ref, o_ref):
    o_ref[...] = acc_ref[0] + acc_ref[1] + lp_ref[...] + b_ref[...]


def _make_xform_first(di, do):
    grid = (_N // _BM,)
    return pl.pallas_call(
        _xform_first_body,
        grid=grid,
        in_specs=[
            pl.BlockSpec((_BM, di), lambda i: (i, 0)),
            pl.BlockSpec((_B, di, do), lambda i: (0, 0, 0)),
            pl.BlockSpec(memory_space=pltpu.SMEM),
            pl.BlockSpec((di, do), lambda i: (0, 0)),
        ],
        out_specs=[
            pl.BlockSpec((_R, _BM, do), lambda i: (0, i, 0)),
            pl.BlockSpec((_BM, do), lambda i: (i, 0)),
        ],
        out_shape=[
            jax.ShapeDtypeStruct((_R, _N, do), jnp.float32),
            jax.ShapeDtypeStruct((_N, do), jnp.float32),
        ],
    )


def _make_xform_mid(di, do):
    grid = (_N // _BM,)
    return pl.pallas_call(
        _xform_mid_body,
        grid=grid,
        in_specs=[
            pl.BlockSpec((_NC, _BM, di), lambda i: (0, i, 0)),
            pl.BlockSpec((_BM, di), lambda i: (i, 0)),
            pl.BlockSpec((1, di), lambda i: (0, 0)),
            pl.BlockSpec((_B, di, do), lambda i: (0, 0, 0)),
            pl.BlockSpec(memory_space=pltpu.SMEM),
            pl.BlockSpec((di, do), lambda i: (0, 0)),
        ],
        out_specs=[
            pl.BlockSpec((_R, _BM, do), lambda i: (0, i, 0)),
            pl.BlockSpec((_BM, do), lambda i: (i, 0)),
        ],
        out_shape=[
            jax.ShapeDtypeStruct((_R, _N, do), jnp.float32),
            jax.ShapeDtypeStruct((_N, do), jnp.float32),
        ],
    )


def _make_final(do):
    grid = (_N // _BM,)
    return pl.pallas_call(
        _final_body,
        grid=grid,
        in_specs=[
            pl.BlockSpec((_NC, _BM, do), lambda i: (0, i, 0)),
            pl.BlockSpec((_BM, do), lambda i: (i, 0)),
            pl.BlockSpec((1, do), lambda i: (0, 0)),
        ],
        out_specs=pl.BlockSpec((_BM, do), lambda i: (i, 0)),
        out_shape=jax.ShapeDtypeStruct((_N, do), jnp.float32),
    )


def _make_sc_agg(do):
    mesh = plsc.VectorSubcoreMesh(core_axis_name="c", subcore_axis_name="s")
    nsl = do // 16

    @functools.partial(
        pl.kernel,
        out_type=jax.ShapeDtypeStruct((_NC, _N, do), jnp.float32),
        mesh=mesh,
        scratch_types=[
            pltpu.VMEM((_K,), jnp.int32),
            pltpu.VMEM((_K,), jnp.int32),
            pltpu.VMEM((_K,), jnp.float32),
            pltpu.VMEM((_K, do), jnp.float32),
            pltpu.VMEM_SHARED((_N, do), jnp.float32),
            pltpu.SemaphoreType.DMA,
        ],
    )
    def agg(table, idxs, dsts, norms, zeros, out, idx_v, dst_v, norm_v, rows_v,
            acc_sh, sem):
        c = lax.axis_index("c")
        s = lax.axis_index("s")
        wid = s * _NC + c

        pltpu.sync_copy(zeros.at[pl.ds(s * _RPT, _RPT)],
                        acc_sh.at[pl.ds(s * _RPT, _RPT)])
        plsc.subcore_barrier()

        base = wid * _EPW

        def chunk(i, carry):
            off = base + i * _K
            pltpu.sync_copy(idxs.at[pl.ds(off, _K)], idx_v)
            pltpu.sync_copy(dsts.at[pl.ds(off, _K)], dst_v)
            pltpu.sync_copy(norms.at[pl.ds(off, _K)], norm_v)
            pltpu.async_copy(table.at[idx_v], rows_v, sem).wait()
            for i_row in range(_K):
                sn = norm_v[i_row]
                for j in range(nsl):
                    sl = pl.ds(j * 16, 16)
                    rows_v[i_row, sl] = rows_v[i_row, sl] * sn
            pltpu.sync_copy(rows_v, acc_sh.at[dst_v], add=True)
            return carry

        lax.fori_loop(0, _NCHUNK, chunk, 0)

        plsc.subcore_barrier()
        pltpu.sync_copy(acc_sh.at[pl.ds(s * _RPT, _RPT)],
                        out.at[c, pl.ds(s * _RPT, _RPT)])

    return agg


_xform0 = _make_xform_first(128, 128)
_xform1 = _make_xform_mid(128, 128)
_xform2 = _make_xform_mid(128, 16)
_final = _make_final(16)
_sc_agg_128 = _make_sc_agg(128)
_sc_agg_16 = _make_sc_agg(16)


def kernel(x, edge_index, edge_type, edge_norm,
           W0, C0, LW0, b0, W1, C1, LW1, b1, W2, C2, LW2, b2):
    src = edge_index[0].astype(jnp.int32)
    dst = edge_index[1].astype(jnp.int32)
    et = edge_type.astype(jnp.int32)
    flat_idx = et * _N + src
    norm = edge_norm.reshape(-1).astype(jnp.float32)
    z128 = jnp.zeros((_N, 128), jnp.float32)
    z16 = jnp.zeros((_N, 16), jnp.float32)

    t0, lp0 = _xform0(x, W0, C0, LW0)
    acc0 = _sc_agg_128(t0.reshape(_R * _N, 128), flat_idx, dst, norm, z128)

    t1, lp1 = _xform1(acc0, lp0, b0.reshape(1, -1), W1, C1, LW1)
    acc1 = _sc_agg_128(t1.reshape(_R * _N, 128), flat_idx, dst, norm, z128)

    t2, lp2 = _xform2(acc1, lp1, b1.reshape(1, -1), W2, C2, LW2)
    acc2 = _sc_agg_16(t2.reshape(_R * _N, 16), flat_idx, dst, norm, z16)

    return _final(acc2, lp2, b2.reshape(1, -1))


# speedup vs baseline: 14.8203x; 14.8203x over previous
import functools

import jax
import jax.numpy as jnp
from jax import lax
from jax.experimental import pallas as pl
from jax.experimental.pallas import tpu as pltpu
from jax.experimental.pallas import tpu_sc as plsc

_N = 10000
_E = 320000
_R = 8
_B = 4

_NC = 2
_NS = 16
_NW = _NC * _NS
_EPW = _E // _NW
_K = 80
_NCHUNK = _EPW // _K
_RPT = 624
_RREM = _N - _NS * _RPT

_BM = 1000


def _emit_transform(h, w_ref, c_ref, lw_ref, t_ref, loop_ref):
    bases = [
        jnp.dot(h, w_ref[b], preferred_element_type=jnp.float32)
        for b in range(_B)
    ]
    for r in range(_R):
        acc = c_ref[r, 0] * bases[0]
        for b in range(1, _B):
            acc = acc + c_ref[r, b] * bases[b]
        t_ref[r] = acc
    loop_ref[...] = jnp.dot(h, lw_ref[...], preferred_element_type=jnp.float32)


def _xform_first_body(x_ref, w_ref, c_ref, lw_ref, t_ref, loop_ref):
    _emit_transform(x_ref[...], w_ref, c_ref, lw_ref, t_ref, loop_ref)


def _xform_mid_body(acc_ref, lp_ref, b_ref, w_ref, c_ref, lw_ref, t_ref, loop_ref):
    h = jnp.maximum(acc_ref[0] + acc_ref[1] + lp_ref[...] + b_ref[...], 0.0)
    _emit_transform(h, w_ref, c_ref, lw_ref, t_ref, loop_ref)


def _final_body(acc_ref, lp_ref, b_ref, o_ref):
    o_ref[...] = acc_ref[0] + acc_ref[1] + lp_ref[...] + b_ref[...]


def _make_xform_first(di, do):
    grid = (_N // _BM,)
    return pl.pallas_call(
        _xform_first_body,
        grid=grid,
        in_specs=[
            pl.BlockSpec((_BM, di), lambda i: (i, 0)),
            pl.BlockSpec((_B, di, do), lambda i: (0, 0, 0)),
            pl.BlockSpec(memory_space=pltpu.SMEM),
            pl.BlockSpec((di, do), lambda i: (0, 0)),
        ],
        out_specs=[
            pl.BlockSpec((_R, _BM, do), lambda i: (0, i, 0)),
            pl.BlockSpec((_BM, do), lambda i: (i, 0)),
        ],
        out_shape=[
            jax.ShapeDtypeStruct((_R, _N, do), jnp.float32),
            jax.ShapeDtypeStruct((_N, do), jnp.float32),
        ],
    )


def _make_xform_mid(di, do):
    grid = (_N // _BM,)
    return pl.pallas_call(
        _xform_mid_body,
        grid=grid,
        in_specs=[
            pl.BlockSpec((_NC, _BM, di), lambda i: (0, i, 0)),
            pl.BlockSpec((_BM, di), lambda i: (i, 0)),
            pl.BlockSpec((1, di), lambda i: (0, 0)),
            pl.BlockSpec((_B, di, do), lambda i: (0, 0, 0)),
            pl.BlockSpec(memory_space=pltpu.SMEM),
            pl.BlockSpec((di, do), lambda i: (0, 0)),
        ],
        out_specs=[
            pl.BlockSpec((_R, _BM, do), lambda i: (0, i, 0)),
            pl.BlockSpec((_BM, do), lambda i: (i, 0)),
        ],
        out_shape=[
            jax.ShapeDtypeStruct((_R, _N, do), jnp.float32),
            jax.ShapeDtypeStruct((_N, do), jnp.float32),
        ],
    )


def _make_final(do):
    grid = (_N // _BM,)
    return pl.pallas_call(
        _final_body,
        grid=grid,
        in_specs=[
            pl.BlockSpec((_NC, _BM, do), lambda i: (0, i, 0)),
            pl.BlockSpec((_BM, do), lambda i: (i, 0)),
            pl.BlockSpec((1, do), lambda i: (0, 0)),
        ],
        out_specs=pl.BlockSpec((_BM, do), lambda i: (i, 0)),
        out_shape=jax.ShapeDtypeStruct((_N, do), jnp.float32),
    )


def _make_sc_agg(do):
    mesh = plsc.VectorSubcoreMesh(core_axis_name="c", subcore_axis_name="s",
                                  num_cores=_NC, num_subcores=_NS)
    nsl = do // 16

    @functools.partial(
        pl.kernel,
        out_type=jax.ShapeDtypeStruct((_NC, _N, do), jnp.float32),
        mesh=mesh,
        scratch_types=[
            pltpu.VMEM((_K,), jnp.int32),
            pltpu.VMEM((_K,), jnp.int32),
            pltpu.VMEM((_K,), jnp.float32),
            pltpu.VMEM((_K, do), jnp.float32),
            pltpu.VMEM_SHARED((_N, do), jnp.float32),
            pltpu.SemaphoreType.DMA,
        ],
        compiler_params=pltpu.CompilerParams(use_tc_tiling_on_sc=False),
    )
    def agg(table, idxs, dsts, norms, zeros, out, idx_v, dst_v, norm_v, rows_v,
            acc_sh, sem):
        c = lax.axis_index("c")
        s = lax.axis_index("s")
        wid = s * _NC + c

        pltpu.sync_copy(zeros.at[pl.ds(s * _RPT, _RPT)],
                        acc_sh.at[pl.ds(s * _RPT, _RPT)])

        @pl.when(s == _NS - 1)
        def _zero_rem():
            pltpu.sync_copy(zeros.at[pl.ds(_NS * _RPT, _RREM)],
                            acc_sh.at[pl.ds(_NS * _RPT, _RREM)])

        plsc.subcore_barrier()

        base = wid * _EPW

        def chunk(i, carry):
            off = base + i * _K
            pltpu.sync_copy(idxs.at[pl.ds(off, _K)], idx_v)
            pltpu.sync_copy(dsts.at[pl.ds(off, _K)], dst_v)
            pltpu.sync_copy(norms.at[pl.ds(off, _K)], norm_v)
            pltpu.async_copy(table.at[idx_v], rows_v, sem).wait()
            for g in range(_K // 16):
                nv = norm_v[pl.ds(g * 16, 16)]
                for t in range(16):
                    i_row = g * 16 + t
                    sn = nv[t]
                    for j in range(nsl):
                        sl = pl.ds(j * 16, 16)
                        rows_v[i_row, sl] = rows_v[i_row, sl] * sn
            pltpu.sync_copy(rows_v, acc_sh.at[dst_v], add=True)
            return carry

        lax.fori_loop(0, _NCHUNK, chunk, 0)

        plsc.subcore_barrier()
        pltpu.sync_copy(acc_sh.at[pl.ds(s * _RPT, _RPT)],
                        out.at[c, pl.ds(s * _RPT, _RPT)])

        @pl.when(s == _NS - 1)
        def _out_rem():
            pltpu.sync_copy(acc_sh.at[pl.ds(_NS * _RPT, _RREM)],
                            out.at[c, pl.ds(_NS * _RPT, _RREM)])

    return agg


_xform0 = _make_xform_first(128, 128)
_xform1 = _make_xform_mid(128, 128)
_xform2 = _make_xform_mid(128, 16)
_final = _make_final(16)
_make_sc_agg = functools.lru_cache(maxsize=None)(_make_sc_agg)


def kernel(x, edge_index, edge_type, edge_norm,
           W0, C0, LW0, b0, W1, C1, LW1, b1, W2, C2, LW2, b2):
    src = edge_index[0].astype(jnp.int32)
    dst = edge_index[1].astype(jnp.int32)
    et = edge_type.astype(jnp.int32)
    flat_idx = et * _N + src
    norm = edge_norm.reshape(-1).astype(jnp.float32)
    z128 = jnp.zeros((_N, 128), jnp.float32)
    z16 = jnp.zeros((_N, 16), jnp.float32)

    sc_agg_128 = _make_sc_agg(128)
    sc_agg_16 = _make_sc_agg(16)

    t0, lp0 = _xform0(x, W0, C0, LW0)
    acc0 = sc_agg_128(t0.reshape(_R * _N, 128), flat_idx, dst, norm, z128)

    t1, lp1 = _xform1(acc0, lp0, b0.reshape(1, -1), W1, C1, LW1)
    acc1 = sc_agg_128(t1.reshape(_R * _N, 128), flat_idx, dst, norm, z128)

    t2, lp2 = _xform2(acc1, lp1, b1.reshape(1, -1), W2, C2, LW2)
    acc2 = sc_agg_16(t2.reshape(_R * _N, 16), flat_idx, dst, norm, z16)

    return _final(acc2, lp2, b2.reshape(1, -1))


# speedup vs baseline: 31.7597x; 2.1430x over previous
import functools

import jax
import jax.numpy as jnp
from jax import lax
from jax.experimental import pallas as pl
from jax.experimental.pallas import tpu as pltpu
from jax.experimental.pallas import tpu_sc as plsc

_N = 10000
_E = 320000
_R = 8
_B = 4

_NC = 2
_NS = 16
_NW = _NC * _NS
_EPW = _E // _NW
_K = 80
_NCHUNK = _EPW // _K
_RPT = 624
_RREM = _N - _NS * _RPT
_CPT = _EPW // _K

_BM = 1000


def _emit_transform(h, w_ref, c_ref, lw_ref, t_ref, loop_ref):
    bases = [
        jnp.dot(h, w_ref[b], preferred_element_type=jnp.float32)
        for b in range(_B)
    ]
    for r in range(_R):
        acc = c_ref[r, 0] * bases[0]
        for b in range(1, _B):
            acc = acc + c_ref[r, b] * bases[b]
        t_ref[r] = acc
    loop_ref[...] = jnp.dot(h, lw_ref[...], preferred_element_type=jnp.float32)


def _xform_first_body(x_ref, w_ref, c_ref, lw_ref, t_ref, loop_ref):
    _emit_transform(x_ref[...], w_ref, c_ref, lw_ref, t_ref, loop_ref)


def _xform_mid_body(acc_ref, lp_ref, b_ref, w_ref, c_ref, lw_ref, t_ref, loop_ref):
    h = jnp.maximum(acc_ref[0] + acc_ref[1] + lp_ref[...] + b_ref[...], 0.0)
    _emit_transform(h, w_ref, c_ref, lw_ref, t_ref, loop_ref)


def _final_body(acc_ref, lp_ref, b_ref, o_ref):
    o_ref[...] = acc_ref[0] + acc_ref[1] + lp_ref[...] + b_ref[...]


def _make_xform_first(di, do):
    grid = (_N // _BM,)
    return pl.pallas_call(
        _xform_first_body,
        grid=grid,
        in_specs=[
            pl.BlockSpec((_BM, di), lambda i: (i, 0)),
            pl.BlockSpec((_B, di, do), lambda i: (0, 0, 0)),
            pl.BlockSpec(memory_space=pltpu.SMEM),
            pl.BlockSpec((di, do), lambda i: (0, 0)),
        ],
        out_specs=[
            pl.BlockSpec((_R, _BM, do), lambda i: (0, i, 0)),
            pl.BlockSpec((_BM, do), lambda i: (i, 0)),
        ],
        out_shape=[
            jax.ShapeDtypeStruct((_R, _N, do), jnp.float32),
            jax.ShapeDtypeStruct((_N, do), jnp.float32),
        ],
    )


def _make_xform_mid(di, do):
    grid = (_N // _BM,)
    return pl.pallas_call(
        _xform_mid_body,
        grid=grid,
        in_specs=[
            pl.BlockSpec((_NC, _BM, di), lambda i: (0, i, 0)),
            pl.BlockSpec((_BM, di), lambda i: (i, 0)),
            pl.BlockSpec((1, di), lambda i: (0, 0)),
            pl.BlockSpec((_B, di, do), lambda i: (0, 0, 0)),
            pl.BlockSpec(memory_space=pltpu.SMEM),
            pl.BlockSpec((di, do), lambda i: (0, 0)),
        ],
        out_specs=[
            pl.BlockSpec((_R, _BM, do), lambda i: (0, i, 0)),
            pl.BlockSpec((_BM, do), lambda i: (i, 0)),
        ],
        out_shape=[
            jax.ShapeDtypeStruct((_R, _N, do), jnp.float32),
            jax.ShapeDtypeStruct((_N, do), jnp.float32),
        ],
    )


def _make_final(do):
    grid = (_N // _BM,)
    return pl.pallas_call(
        _final_body,
        grid=grid,
        in_specs=[
            pl.BlockSpec((_NC, _BM, do), lambda i: (0, i, 0)),
            pl.BlockSpec((_BM, do), lambda i: (i, 0)),
            pl.BlockSpec((1, do), lambda i: (0, 0)),
        ],
        out_specs=pl.BlockSpec((_BM, do), lambda i: (i, 0)),
        out_shape=jax.ShapeDtypeStruct((_N, do), jnp.float32),
    )


def _make_sc_agg(do):
    mesh = plsc.VectorSubcoreMesh(core_axis_name="c", subcore_axis_name="s",
                                  num_cores=_NC, num_subcores=_NS)
    nsl = do // 16

    @functools.partial(
        pl.kernel,
        out_type=jax.ShapeDtypeStruct((_NC, _N, do), jnp.float32),
        mesh=mesh,
        scratch_types=[
            pltpu.VMEM((_CPT, _K), jnp.int32),
            pltpu.VMEM((_CPT, _K), jnp.int32),
            pltpu.VMEM((_CPT, _K), jnp.float32),
            pltpu.VMEM((_K, do), jnp.float32),
            pltpu.VMEM((_K, do), jnp.float32),
            pltpu.VMEM_SHARED((_N, do), jnp.float32),
            pltpu.SemaphoreType.DMA,
            pltpu.SemaphoreType.DMA,
            pltpu.SemaphoreType.DMA,
            pltpu.SemaphoreType.DMA,
        ],
        compiler_params=pltpu.CompilerParams(use_tc_tiling_on_sc=False),
    )
    def agg(table, idxs, dsts, norms, zeros, out, idx_v, dst_v, norm_v,
            rows0, rows1, acc_sh, gsem0, gsem1, ssem0, ssem1):
        c = lax.axis_index("c")
        s = lax.axis_index("s")
        wid = s * _NC + c

        rowbase = wid * _CPT
        pltpu.sync_copy(idxs.at[pl.ds(rowbase, _CPT)], idx_v)
        pltpu.sync_copy(dsts.at[pl.ds(rowbase, _CPT)], dst_v)
        pltpu.sync_copy(norms.at[pl.ds(rowbase, _CPT)], norm_v)

        pltpu.sync_copy(zeros.at[pl.ds(s * _RPT, _RPT)],
                        acc_sh.at[pl.ds(s * _RPT, _RPT)])

        @pl.when(s == _NS - 1)
        def _zero_rem():
            pltpu.sync_copy(zeros.at[pl.ds(_NS * _RPT, _RREM)],
                            acc_sh.at[pl.ds(_NS * _RPT, _RREM)])

        plsc.subcore_barrier()

        rows = (rows0, rows1)
        gsem = (gsem0, gsem1)
        ssem = (ssem0, ssem1)

        def issue_gather(ci, b):
            pltpu.async_copy(table.at[idx_v.at[ci]], rows[b], gsem[b])

        def wait_gather(b):
            pltpu.make_async_copy(table.at[idx_v.at[0]], rows[b],
                                  gsem[b]).wait()

        def issue_scatter(ci, b):
            pltpu.async_copy(rows[b], acc_sh.at[dst_v.at[ci]], ssem[b],
                             add=True)

        def wait_scatter(b):
            pltpu.make_async_copy(rows[b], acc_sh.at[dst_v.at[0]],
                                  ssem[b]).wait()

        def scale(ci, b):
            buf = rows[b]
            for g in range(_K // 16):
                nv = norm_v[ci, pl.ds(g * 16, 16)]
                for t in range(16):
                    i_row = g * 16 + t
                    sn = nv[t]
                    for j in range(nsl):
                        sl = pl.ds(j * 16, 16)
                        buf[i_row, sl] = buf[i_row, sl] * sn

        issue_gather(0, 0)

        def pair(j, carry):
            c0 = j * 2
            wait_gather(0)

            @pl.when(c0 >= 1)
            def _():
                wait_scatter(1)

            issue_gather(c0 + 1, 1)
            scale(c0, 0)
            issue_scatter(c0, 0)
            wait_gather(1)
            wait_scatter(0)
            issue_gather(c0 + 2, 0)
            scale(c0 + 1, 1)
            issue_scatter(c0 + 1, 1)
            return carry

        lax.fori_loop(0, (_CPT - 1) // 2, pair, 0)

        wait_gather(0)
        wait_scatter(1)
        scale(_CPT - 1, 0)
        pltpu.sync_copy(rows0, acc_sh.at[dst_v.at[_CPT - 1]], add=True)

        plsc.subcore_barrier()
        pltpu.sync_copy(acc_sh.at[pl.ds(s * _RPT, _RPT)],
                        out.at[c, pl.ds(s * _RPT, _RPT)])

        @pl.when(s == _NS - 1)
        def _out_rem():
            pltpu.sync_copy(acc_sh.at[pl.ds(_NS * _RPT, _RREM)],
                            out.at[c, pl.ds(_NS * _RPT, _RREM)])

    return agg


_xform0 = _make_xform_first(128, 128)
_xform1 = _make_xform_mid(128, 128)
_xform2 = _make_xform_mid(128, 16)
_final = _make_final(16)
_make_sc_agg = functools.lru_cache(maxsize=None)(_make_sc_agg)


def kernel(x, edge_index, edge_type, edge_norm,
           W0, C0, LW0, b0, W1, C1, LW1, b1, W2, C2, LW2, b2):
    src = edge_index[0].astype(jnp.int32)
    dst = edge_index[1].astype(jnp.int32)
    et = edge_type.astype(jnp.int32)
    flat_idx = (et * _N + src).reshape(_E // _K, _K)
    dst = dst.reshape(_E // _K, _K)
    norm = edge_norm.reshape(_E // _K, _K).astype(jnp.float32)
    z128 = jnp.zeros((_N, 128), jnp.float32)
    z16 = jnp.zeros((_N, 16), jnp.float32)

    sc_agg_128 = _make_sc_agg(128)
    sc_agg_16 = _make_sc_agg(16)

    t0, lp0 = _xform0(x, W0, C0, LW0)
    acc0 = sc_agg_128(t0.reshape(_R * _N, 128), flat_idx, dst, norm, z128)

    t1, lp1 = _xform1(acc0, lp0, b0.reshape(1, -1), W1, C1, LW1)
    acc1 = sc_agg_128(t1.reshape(_R * _N, 128), flat_idx, dst, norm, z128)

    t2, lp2 = _xform2(acc1, lp1, b1.reshape(1, -1), W2, C2, LW2)
    acc2 = sc_agg_16(t2.reshape(_R * _N, 16), flat_idx, dst, norm, z16)

    return _final(acc2, lp2, b2.reshape(1, -1))
